# baseline (speedup 1.0000x reference)
import functools

import jax
import jax.numpy as jnp
from jax.experimental import pallas as pl
from jax.experimental.pallas import tpu as pltpu

_K = 128
_D = 32
_L = 8
_B = 4096
_N = _B * _L
_ROWS = 1024
_GRID = _N // _ROWS
_BB = _ROWS // _L


def _body(x_ref, cb_ref, pol_ref, qnt_ref, cset_ref):
    x = x_ref[...]
    cb = cb_ref[...]
    prod = jax.lax.dot_general(
        x, cb, (((1,), (1,)), ((), ())),
        preferred_element_type=jnp.float32)
    dist = (jnp.sum(x * x, axis=1, keepdims=True)
            + jnp.sum(cb * cb, axis=1)[None, :]) - 2.0 * prod
    mins = jnp.min(dist, axis=1, keepdims=True)
    iota = jax.lax.broadcasted_iota(jnp.int32, dist.shape, 1)
    idx = jnp.min(jnp.where(dist == mins, iota, _K), axis=1, keepdims=True)
    onehot = (iota == idx).astype(jnp.float32)
    q = jax.lax.dot_general(
        onehot, cb, (((1,), (0,)), ((), ())),
        preferred_element_type=jnp.float32)
    pol_ref[...] = q
    qnt_ref[...] = q
    cset_ref[...] = jnp.broadcast_to(cb[None, :, :], cset_ref.shape)


@functools.partial(jax.jit, static_argnames=())
def kernel(latent, codebook):
    flat = latent.reshape(_N, _D)
    pol, qnt, cset = pl.pallas_call(
        _body,
        grid=(_GRID,),
        in_specs=[
            pl.BlockSpec((_ROWS, _D), lambda i: (i, 0)),
            pl.BlockSpec((_K, _D), lambda i: (0, 0)),
        ],
        out_specs=[
            pl.BlockSpec((_ROWS, _D), lambda i: (i, 0)),
            pl.BlockSpec((_ROWS, _D), lambda i: (i, 0)),
            pl.BlockSpec((_BB, _K, _D), lambda i: (i, 0, 0)),
        ],
        out_shape=[
            jax.ShapeDtypeStruct((_N, _D), jnp.float32),
            jax.ShapeDtypeStruct((_N, _D), jnp.float32),
            jax.ShapeDtypeStruct((_B, _K, _D), jnp.float32),
        ],
        compiler_params=pltpu.CompilerParams(
            dimension_semantics=("arbitrary",),
        ),
    )(flat, codebook)
    shape = latent.shape
    return (pol.reshape(shape), qnt.reshape(shape), cset)


# speedup vs baseline: 7.2242x; 7.2242x over previous
import functools

import jax
import jax.numpy as jnp
from jax.experimental import pallas as pl
from jax.experimental.pallas import tpu as pltpu

_K = 128
_D = 32
_L = 8
_B = 4096
_NB = 1024
_JG = _B // _NB
_BB = _B // (_L * _JG)


def _body(x_ref, cb_ref, cbt_ref, pol_ref, qnt_ref, cset_ref):
    x = x_ref[0]
    cb = cb_ref[...]
    cbt = cbt_ref[...]
    prod = jax.lax.dot_general(
        cb, x, (((1,), (0,)), ((), ())),
        preferred_element_type=jnp.float32)
    dist = (jnp.sum(x * x, axis=0, keepdims=True)
            + jnp.sum(cb * cb, axis=1, keepdims=True)) - 2.0 * prod
    mins = jnp.min(dist, axis=0, keepdims=True)
    iota = jax.lax.broadcasted_iota(jnp.int32, dist.shape, 0)
    idx = jnp.min(jnp.where(dist == mins, iota, _K), axis=0, keepdims=True)
    onehot = (iota == idx).astype(jnp.float32)
    q = jax.lax.dot_general(
        cbt, onehot, (((1,), (0,)), ((), ())),
        preferred_element_type=jnp.float32)
    pol_ref[0] = q
    qnt_ref[0] = q
    cset_ref[...] = jnp.broadcast_to(cbt[None], cset_ref.shape)


@functools.partial(jax.jit, static_argnames=())
def kernel(latent, codebook):
    lat_t = latent.transpose(1, 2, 0)
    cbt = codebook.T
    pol, qnt, cset_t = pl.pallas_call(
        _body,
        grid=(_L, _JG),
        in_specs=[
            pl.BlockSpec((1, _D, _NB), lambda l, j: (l, 0, j)),
            pl.BlockSpec((_K, _D), lambda l, j: (0, 0)),
            pl.BlockSpec((_D, _K), lambda l, j: (0, 0)),
        ],
        out_specs=[
            pl.BlockSpec((1, _D, _NB), lambda l, j: (l, 0, j)),
            pl.BlockSpec((1, _D, _NB), lambda l, j: (l, 0, j)),
            pl.BlockSpec((_BB, _D, _K), lambda l, j: (j * _L + l, 0, 0)),
        ],
        out_shape=[
            jax.ShapeDtypeStruct((_L, _D, _B), jnp.float32),
            jax.ShapeDtypeStruct((_L, _D, _B), jnp.float32),
            jax.ShapeDtypeStruct((_B, _D, _K), jnp.float32),
        ],
        compiler_params=pltpu.CompilerParams(
            dimension_semantics=("arbitrary", "arbitrary"),
        ),
    )(lat_t, codebook, cbt)
    pol = pol.transpose(2, 0, 1)
    qnt = qnt.transpose(2, 0, 1)
    return (pol, qnt, cset_t.transpose(0, 2, 1))


# speedup vs baseline: 9.7963x; 1.3561x over previous
import functools

import jax
import jax.numpy as jnp
from jax.experimental import pallas as pl
from jax.experimental.pallas import tpu as pltpu

_K = 128
_D = 32
_L = 8
_B = 4096
_NB = 4096
_JG = _B // _NB
_BB = _B // (_L * _JG)


def _body(x_ref, cb_ref, cbt_ref, pol_ref, qnt_ref, cset_ref):
    x = x_ref[0]
    cb = cb_ref[...]
    cbt = cbt_ref[...]
    prod = jax.lax.dot_general(
        cb, x, (((1,), (0,)), ((), ())),
        preferred_element_type=jnp.float32)
    dist = (jnp.sum(x * x, axis=0, keepdims=True)
            + jnp.sum(cb * cb, axis=1, keepdims=True)) - 2.0 * prod
    mins = jnp.min(dist, axis=0, keepdims=True)
    iota = jax.lax.broadcasted_iota(jnp.int32, dist.shape, 0)
    idx = jnp.min(jnp.where(dist == mins, iota, _K), axis=0, keepdims=True)
    onehot = (iota == idx).astype(jnp.float32)
    q = jax.lax.dot_general(
        cbt, onehot, (((1,), (0,)), ((), ())),
        preferred_element_type=jnp.float32)
    pol_ref[0] = q
    qnt_ref[0] = q
    cset_ref[...] = jnp.broadcast_to(cbt[None], cset_ref.shape)


@functools.partial(jax.jit, static_argnames=())
def kernel(latent, codebook):
    lat_t = latent.transpose(1, 2, 0)
    cbt = codebook.T
    pol, qnt, cset_t = pl.pallas_call(
        _body,
        grid=(_L,),
        in_specs=[
            pl.BlockSpec((1, _D, _NB), lambda l: (l, 0, 0)),
            pl.BlockSpec((_K, _D), lambda l: (0, 0)),
            pl.BlockSpec((_D, _K), lambda l: (0, 0)),
        ],
        out_specs=[
            pl.BlockSpec((1, _D, _NB), lambda l: (l, 0, 0)),
            pl.BlockSpec((1, _D, _NB), lambda l: (l, 0, 0)),
            pl.BlockSpec((_BB, _D, _K), lambda l: (l, 0, 0)),
        ],
        out_shape=[
            jax.ShapeDtypeStruct((_L, _D, _B), jnp.float32),
            jax.ShapeDtypeStruct((_L, _D, _B), jnp.float32),
            jax.ShapeDtypeStruct((_B, _D, _K), jnp.float32),
        ],
        compiler_params=pltpu.CompilerParams(
            dimension_semantics=("arbitrary",),
        ),
    )(lat_t, codebook, cbt)
    pol = pol.transpose(2, 0, 1)
    qnt = qnt.transpose(2, 0, 1)
    return (pol, qnt, cset_t.transpose(0, 2, 1))
